# baseline (speedup 1.0000x reference)
import functools

import jax
import jax.numpy as jnp
from jax import lax
from jax.experimental import pallas as pl
from jax.experimental.pallas import tpu as pltpu
from jax.experimental.pallas import tpu_sc as plsc

HIDDEN = 1024
E = 8
DFF = 2048
TOK = 2048
C = 512
PADC = 520
NC = 2
NS = 16
NW = NC * NS
TPW = TOK // NW
CHUNK = 32
BLK = 256


def _gating_body(x_ref, wg_ref, s1_ref, s2_ref, g1_ref, g2_ref,
                 w1b_ref, w2b_ref, stats_ref, laux_ref):
    x = x_ref[...]
    wg = wg_ref[...]
    logits = jnp.dot(x, wg, preferred_element_type=jnp.float32)
    m = jnp.max(logits, axis=1, keepdims=True)
    p = jnp.exp(logits - m)
    gates = p / jnp.sum(p, axis=1, keepdims=True)

    cols = lax.broadcasted_iota(jnp.int32, (TOK, E), 1)
    gmax = jnp.max(gates, axis=1, keepdims=True)
    idx1 = jnp.min(jnp.where(gates == gmax, cols, E), axis=1, keepdims=True)
    sel1 = cols == idx1
    mask1 = sel1.astype(jnp.float32)
    lm = jnp.where(sel1, -jnp.inf, logits)
    m2 = jnp.max(lm, axis=1, keepdims=True)
    idx2 = jnp.min(jnp.where(lm == m2, cols, E), axis=1, keepdims=True)
    sel2 = cols == idx2
    mask2 = sel2.astype(jnp.float32)

    tri = (lax.broadcasted_iota(jnp.int32, (BLK, BLK), 0)
           >= lax.broadcasted_iota(jnp.int32, (BLK, BLK), 1)).astype(jnp.float32)
    run1 = jnp.zeros((1, E), jnp.float32)
    run2 = jnp.zeros((1, E), jnp.float32)
    l1s, l2s = [], []
    for b in range(TOK // BLK):
        b1 = mask1[b * BLK:(b + 1) * BLK]
        b2 = mask2[b * BLK:(b + 1) * BLK]
        l1s.append(jnp.dot(tri, b1, preferred_element_type=jnp.float32) + run1)
        l2s.append(jnp.dot(tri, b2, preferred_element_type=jnp.float32) + run2)
        run1 = run1 + jnp.sum(b1, axis=0, keepdims=True)
        run2 = run2 + jnp.sum(b2, axis=0, keepdims=True)
    n1, n2 = run1, run2
    loc1 = jnp.concatenate(l1s, axis=0) - 1.0
    loc2 = jnp.concatenate(l2s, axis=0) - 1.0 + n1

    loc1_sel = jnp.sum(loc1 * mask1, axis=1, keepdims=True)
    loc2_sel = jnp.sum(loc2 * mask2, axis=1, keepdims=True)
    keep1 = (loc1_sel < C).astype(jnp.float32)
    keep2 = (loc2_sel < C).astype(jnp.float32)
    pos1 = (loc1_sel * keep1).astype(jnp.int32)
    pos2 = (loc2_sel * keep2).astype(jnp.int32)
    posc1 = jnp.clip(pos1, 0, C - 1)
    posc2 = jnp.clip(pos2, 0, C - 1)
    gate1 = jnp.sum(gates * mask1, axis=1, keepdims=True) * keep1
    gate2 = jnp.sum(gates * mask2, axis=1, keepdims=True) * keep2
    denom = gate1 + gate2
    denom = jnp.where(denom > 1e-9, denom, 1e-9)
    w1 = gate1 / denom
    w2 = gate2 / denom

    s1 = jnp.where(keep1 > 0, idx1 * PADC + pos1, idx1 * PADC + C)
    s2 = jnp.where(keep2 > 0, idx2 * PADC + pos2, idx2 * PADC + C)
    g1 = idx1 * C + posc1
    g2 = idx2 * C + posc2

    s1_ref[...] = s1
    s2_ref[...] = s2
    g1_ref[...] = g1
    g2_ref[...] = g2
    w1b_ref[...] = jnp.broadcast_to(w1, (TOK, 16))
    w2b_ref[...] = jnp.broadcast_to(w2, (TOK, 16))
    filled = jnp.minimum(n1 + n2, C)
    stats_ref[...] = jnp.concatenate([filled, n1], axis=0).astype(jnp.int32)
    me = jnp.mean(gates, axis=0, keepdims=True)
    laux_ref[...] = (jnp.sum(me * (n1 / TOK)) * E).reshape(1, 1)


def _gating(x, wg):
    return pl.pallas_call(
        _gating_body,
        out_shape=(
            jax.ShapeDtypeStruct((TOK, 1), jnp.int32),
            jax.ShapeDtypeStruct((TOK, 1), jnp.int32),
            jax.ShapeDtypeStruct((TOK, 1), jnp.int32),
            jax.ShapeDtypeStruct((TOK, 1), jnp.int32),
            jax.ShapeDtypeStruct((TOK, 16), jnp.float32),
            jax.ShapeDtypeStruct((TOK, 16), jnp.float32),
            jax.ShapeDtypeStruct((2, E), jnp.int32),
            jax.ShapeDtypeStruct((1, 1), jnp.float32),
        ),
    )(x, wg)


DCH = 32


@functools.lru_cache(maxsize=None)
def _sc_mesh():
    return plsc.VectorSubcoreMesh(
        core_axis_name="c", subcore_axis_name="s",
        num_cores=NC, num_subcores=NS)


@functools.lru_cache(maxsize=None)
def _dispatch_kernel():
    @functools.partial(
        pl.kernel,
        out_type=jax.ShapeDtypeStruct((E * PADC, HIDDEN), jnp.float32),
        mesh=_sc_mesh(),
        scratch_types=[
            [pltpu.VMEM((DCH,), jnp.int32)] * 2,
            [pltpu.VMEM((DCH,), jnp.int32)] * 2,
            [pltpu.VMEM((DCH, HIDDEN), jnp.float32)] * 2,
            [pltpu.SemaphoreType.DMA] * 2,
            [pltpu.SemaphoreType.DMA] * 2,
        ],
    )
    def _dispatch(x_hbm, s1_hbm, s2_hbm, buf_hbm, i1_v, i2_v, rows_v,
                  lsem, ssem):
        wid = lax.axis_index("s") * NC + lax.axis_index("c")
        base = wid * TPW
        for c in range(2):
            pltpu.sync_copy(s1_hbm.at[wid, pl.ds(c * DCH, DCH)], i1_v[c])
            pltpu.sync_copy(s2_hbm.at[wid, pl.ds(c * DCH, DCH)], i2_v[c])
        loads = {0: pltpu.async_copy(
            x_hbm.at[pl.ds(base, DCH)], rows_v[0], lsem[0])}
        scats = []
        for c in range(2):
            if c + 1 < 2:
                loads[c + 1] = pltpu.async_copy(
                    x_hbm.at[pl.ds(base + (c + 1) * DCH, DCH)],
                    rows_v[c + 1], lsem[c + 1])
            loads.pop(c).wait()
            scats.append(pltpu.async_copy(
                rows_v[c], buf_hbm.at[i1_v[c]], ssem[0]))
            scats.append(pltpu.async_copy(
                rows_v[c], buf_hbm.at[i2_v[c]], ssem[1]))
        for cp in scats:
            cp.wait()

    return _dispatch


def _ffn_body(stats_ref, x_ref, w1_ref, w2_ref, y_ref):
    e = pl.program_id(0)
    filled = stats_ref[0, e]
    rows = lax.broadcasted_iota(jnp.int32, (PADC, HIDDEN), 0)
    xm = jnp.where(rows < filled, x_ref[...], 0.0)[:C]
    h = jnp.maximum(
        jnp.dot(xm.astype(jnp.bfloat16), w1_ref[0].astype(jnp.bfloat16),
                preferred_element_type=jnp.float32), 0.0)
    y_ref[...] = jnp.dot(h.astype(jnp.bfloat16), w2_ref[0].astype(jnp.bfloat16),
                         preferred_element_type=jnp.float32)


def _ffn(stats, buf, w1, w2):
    return pl.pallas_call(
        _ffn_body,
        grid=(E,),
        in_specs=[
            pl.BlockSpec(memory_space=pltpu.SMEM),
            pl.BlockSpec((PADC, HIDDEN), lambda e: (e, 0)),
            pl.BlockSpec((1, HIDDEN, DFF), lambda e: (e, 0, 0)),
            pl.BlockSpec((1, DFF, HIDDEN), lambda e: (e, 0, 0)),
        ],
        out_specs=pl.BlockSpec((C, HIDDEN), lambda e: (e, 0)),
        out_shape=jax.ShapeDtypeStruct((E * C, HIDDEN), jnp.float32),
    )(stats, buf, w1, w2)


CCH = 16
NCH = TPW // CCH


@functools.lru_cache(maxsize=None)
def _combine_kernel():
    @functools.partial(
        pl.kernel,
        out_type=jax.ShapeDtypeStruct((TOK, HIDDEN), jnp.float32),
        mesh=_sc_mesh(),
        scratch_types=[
            pltpu.VMEM((TPW,), jnp.int32),
            pltpu.VMEM((TPW,), jnp.int32),
            pltpu.VMEM((TPW, 16), jnp.float32),
            pltpu.VMEM((TPW, 16), jnp.float32),
            [pltpu.VMEM((CCH, HIDDEN), jnp.float32)] * 2,
            [pltpu.VMEM((CCH, HIDDEN), jnp.float32)] * 2,
            [pltpu.VMEM((CCH, HIDDEN), jnp.float32)] * 2,
            [pltpu.SemaphoreType.DMA] * 2,
            [pltpu.SemaphoreType.DMA] * 2,
            [pltpu.SemaphoreType.DMA] * 2,
        ],
    )
    def _combine(y_hbm, g1_hbm, g2_hbm, w1_hbm, w2_hbm, out_hbm,
                 i1_v, i2_v, w1_v, w2_v, a_v, b_v, o_v, sa, sb, so):
        wid = lax.axis_index("s") * NC + lax.axis_index("c")
        base = wid * TPW
        pltpu.sync_copy(g1_hbm.at[wid], i1_v)
        pltpu.sync_copy(g2_hbm.at[wid], i2_v)
        pltpu.sync_copy(w1_hbm.at[wid], w1_v)
        pltpu.sync_copy(w2_hbm.at[wid], w2_v)

        def start_gathers(ch):
            s = ch % 2
            sl = pl.ds(ch * CCH, CCH)
            return (pltpu.async_copy(y_hbm.at[i1_v.at[sl]], a_v[s], sa[s]),
                    pltpu.async_copy(y_hbm.at[i2_v.at[sl]], b_v[s], sb[s]))

        pending = {0: start_gathers(0)}
        stores = {}
        for ch in range(NCH):
            s = ch % 2
            if ch + 1 < NCH:
                pending[ch + 1] = start_gathers(ch + 1)
            ga, gb = pending.pop(ch)
            ga.wait()
            gb.wait()
            if ch >= 2:
                stores.pop(ch - 2).wait()

            def run_chunk(s, ch):
                @plsc.parallel_loop(0, CCH, step=1)
                def _body(t):
                    g1 = w1_v[ch * CCH + t]
                    g2 = w2_v[ch * CCH + t]
                    for j in range(HIDDEN // 16):
                        sl = pl.ds(j * 16, 16)
                        o_v[s][t, sl] = (g1 * a_v[s][t, sl]
                                         + g2 * b_v[s][t, sl])

            run_chunk(s, ch)
            stores[ch] = pltpu.async_copy(
                o_v[s], out_hbm.at[pl.ds(base + ch * CCH, CCH)], so[s])
        for ch in sorted(stores):
            stores.pop(ch).wait()

    return _combine


def kernel(hidden_states, Wg, W1, W2):
    s1, s2, g1, g2, w1b, w2b, stats, laux = _gating(hidden_states, Wg)
    s1r = s1.reshape(NW, TPW)
    s2r = s2.reshape(NW, TPW)
    g1r = g1.reshape(NW, TPW)
    g2r = g2.reshape(NW, TPW)
    w1r = w1b.reshape(NW, TPW, 16)
    w2r = w2b.reshape(NW, TPW, 16)
    buf = _dispatch_kernel()(hidden_states, s1r, s2r)
    y = _ffn(stats, buf, W1, W2)
    out = _combine_kernel()(y, g1r, g2r, w1r, w2r)
    return (out, laux[0, 0], stats[1])


# speedup vs baseline: 1.0044x; 1.0044x over previous
import functools

import jax
import jax.numpy as jnp
from jax import lax
from jax.experimental import pallas as pl
from jax.experimental.pallas import tpu as pltpu
from jax.experimental.pallas import tpu_sc as plsc

HIDDEN = 1024
E = 8
DFF = 2048
TOK = 2048
C = 512
PADC = 520
NC = 2
NS = 16
NW = NC * NS
TPW = TOK // NW
BLK = 256


def _gating_body(x_ref, wg_ref, s1_ref, s2_ref, g1_ref, g2_ref,
                 w1b_ref, w2b_ref, stats_ref, laux_ref):
    x = x_ref[...]
    wg = wg_ref[...]
    logits = jnp.dot(x, wg, preferred_element_type=jnp.float32)
    m = jnp.max(logits, axis=1, keepdims=True)
    p = jnp.exp(logits - m)
    gates = p / jnp.sum(p, axis=1, keepdims=True)

    cols = lax.broadcasted_iota(jnp.int32, (TOK, E), 1)
    gmax = jnp.max(gates, axis=1, keepdims=True)
    idx1 = jnp.min(jnp.where(gates == gmax, cols, E), axis=1, keepdims=True)
    sel1 = cols == idx1
    mask1 = sel1.astype(jnp.float32)
    lm = jnp.where(sel1, -jnp.inf, logits)
    m2 = jnp.max(lm, axis=1, keepdims=True)
    idx2 = jnp.min(jnp.where(lm == m2, cols, E), axis=1, keepdims=True)
    sel2 = cols == idx2
    mask2 = sel2.astype(jnp.float32)

    tri = (lax.broadcasted_iota(jnp.int32, (BLK, BLK), 0)
           >= lax.broadcasted_iota(jnp.int32, (BLK, BLK), 1)).astype(jnp.float32)
    run1 = jnp.zeros((1, E), jnp.float32)
    run2 = jnp.zeros((1, E), jnp.float32)
    l1s, l2s = [], []
    for b in range(TOK // BLK):
        b1 = mask1[b * BLK:(b + 1) * BLK]
        b2 = mask2[b * BLK:(b + 1) * BLK]
        l1s.append(jnp.dot(tri, b1, preferred_element_type=jnp.float32) + run1)
        l2s.append(jnp.dot(tri, b2, preferred_element_type=jnp.float32) + run2)
        run1 = run1 + jnp.sum(b1, axis=0, keepdims=True)
        run2 = run2 + jnp.sum(b2, axis=0, keepdims=True)
    n1, n2 = run1, run2
    loc1 = jnp.concatenate(l1s, axis=0) - 1.0
    loc2 = jnp.concatenate(l2s, axis=0) - 1.0 + n1

    loc1_sel = jnp.sum(loc1 * mask1, axis=1, keepdims=True)
    loc2_sel = jnp.sum(loc2 * mask2, axis=1, keepdims=True)
    keep1 = (loc1_sel < C).astype(jnp.float32)
    keep2 = (loc2_sel < C).astype(jnp.float32)
    pos1 = (loc1_sel * keep1).astype(jnp.int32)
    pos2 = (loc2_sel * keep2).astype(jnp.int32)
    posc1 = jnp.clip(pos1, 0, C - 1)
    posc2 = jnp.clip(pos2, 0, C - 1)
    gate1 = jnp.sum(gates * mask1, axis=1, keepdims=True) * keep1
    gate2 = jnp.sum(gates * mask2, axis=1, keepdims=True) * keep2
    denom = gate1 + gate2
    denom = jnp.where(denom > 1e-9, denom, 1e-9)
    w1 = gate1 / denom
    w2 = gate2 / denom

    s1 = jnp.where(keep1 > 0, idx1 * PADC + pos1, idx1 * PADC + C)
    s2 = jnp.where(keep2 > 0, idx2 * PADC + pos2, idx2 * PADC + C)
    g1 = idx1 * C + posc1
    g2 = idx2 * C + posc2

    s1_ref[...] = s1
    s2_ref[...] = s2
    g1_ref[...] = g1
    g2_ref[...] = g2
    w1b_ref[...] = jnp.broadcast_to(w1, (TOK, 16))
    w2b_ref[...] = jnp.broadcast_to(w2, (TOK, 16))
    filled = jnp.minimum(n1 + n2, C)
    stats_ref[...] = jnp.concatenate([filled, n1], axis=0).astype(jnp.int32)
    me = jnp.mean(gates, axis=0, keepdims=True)
    laux_ref[...] = (jnp.sum(me * (n1 / TOK)) * E).reshape(1, 1)


def _gating(x, wg):
    return pl.pallas_call(
        _gating_body,
        out_shape=(
            jax.ShapeDtypeStruct((TOK, 1), jnp.int32),
            jax.ShapeDtypeStruct((TOK, 1), jnp.int32),
            jax.ShapeDtypeStruct((TOK, 1), jnp.int32),
            jax.ShapeDtypeStruct((TOK, 1), jnp.int32),
            jax.ShapeDtypeStruct((TOK, 16), jnp.float32),
            jax.ShapeDtypeStruct((TOK, 16), jnp.float32),
            jax.ShapeDtypeStruct((2, E), jnp.int32),
            jax.ShapeDtypeStruct((1, 1), jnp.float32),
        ),
    )(x, wg)


@functools.lru_cache(maxsize=None)
def _sc_mesh():
    return plsc.VectorSubcoreMesh(
        core_axis_name="c", subcore_axis_name="s",
        num_cores=NC, num_subcores=NS)


@functools.lru_cache(maxsize=None)
def _dispatch_kernel():
    @functools.partial(
        pl.kernel,
        out_type=jax.ShapeDtypeStruct((E * PADC, HIDDEN), jnp.float32),
        mesh=_sc_mesh(),
        scratch_types=[
            pltpu.VMEM((TPW,), jnp.int32),
            pltpu.VMEM((TPW,), jnp.int32),
            pltpu.VMEM((TPW, HIDDEN), jnp.float32),
            pltpu.SemaphoreType.DMA,
            pltpu.SemaphoreType.DMA,
        ],
    )
    def _dispatch(x_hbm, s1_hbm, s2_hbm, buf_hbm, i1_v, i2_v, rows_v,
                  sem1, sem2):
        wid = lax.axis_index("s") * NC + lax.axis_index("c")
        base = wid * TPW
        pltpu.sync_copy(s1_hbm.at[wid], i1_v)
        pltpu.sync_copy(s2_hbm.at[wid], i2_v)
        pltpu.sync_copy(x_hbm.at[pl.ds(base, TPW)], rows_v)
        cp1 = pltpu.async_copy(rows_v, buf_hbm.at[i1_v], sem1)
        cp2 = pltpu.async_copy(rows_v, buf_hbm.at[i2_v], sem2)
        cp1.wait()
        cp2.wait()

    return _dispatch


def _ffn_body(stats_ref, x_ref, w1_ref, w2_ref, y_ref):
    e = pl.program_id(0)
    filled = stats_ref[0, e]
    rows = lax.broadcasted_iota(jnp.int32, (PADC, HIDDEN), 0)
    xm = jnp.where(rows < filled, x_ref[...], 0.0)[:C]
    h = jnp.maximum(
        jnp.dot(xm.astype(jnp.bfloat16), w1_ref[0].astype(jnp.bfloat16),
                preferred_element_type=jnp.float32), 0.0)
    y_ref[...] = jnp.dot(h.astype(jnp.bfloat16), w2_ref[0].astype(jnp.bfloat16),
                         preferred_element_type=jnp.float32)


def _ffn(stats, buf, w1, w2):
    return pl.pallas_call(
        _ffn_body,
        grid=(E,),
        in_specs=[
            pl.BlockSpec(memory_space=pltpu.SMEM),
            pl.BlockSpec((PADC, HIDDEN), lambda e: (e, 0)),
            pl.BlockSpec((1, HIDDEN, DFF), lambda e: (e, 0, 0)),
            pl.BlockSpec((1, DFF, HIDDEN), lambda e: (e, 0, 0)),
        ],
        out_specs=pl.BlockSpec((C, HIDDEN), lambda e: (e, 0)),
        out_shape=jax.ShapeDtypeStruct((E * C, HIDDEN), jnp.float32),
    )(stats, buf, w1, w2)


CCH = 16
NCH = TPW // CCH


@functools.lru_cache(maxsize=None)
def _combine_kernel():
    @functools.partial(
        pl.kernel,
        out_type=jax.ShapeDtypeStruct((TOK, HIDDEN), jnp.float32),
        mesh=_sc_mesh(),
        scratch_types=[
            pltpu.VMEM((TPW,), jnp.int32),
            pltpu.VMEM((TPW,), jnp.int32),
            pltpu.VMEM((TPW, 16), jnp.float32),
            pltpu.VMEM((TPW, 16), jnp.float32),
            [pltpu.VMEM((CCH, HIDDEN), jnp.float32)] * 2,
            [pltpu.VMEM((CCH, HIDDEN), jnp.float32)] * 2,
            [pltpu.VMEM((CCH, HIDDEN), jnp.float32)] * 2,
            [pltpu.SemaphoreType.DMA] * 2,
            [pltpu.SemaphoreType.DMA] * 2,
            [pltpu.SemaphoreType.DMA] * 2,
        ],
    )
    def _combine(y_hbm, g1_hbm, g2_hbm, w1_hbm, w2_hbm, out_hbm,
                 i1_v, i2_v, w1_v, w2_v, a_v, b_v, o_v, sa, sb, so):
        wid = lax.axis_index("s") * NC + lax.axis_index("c")
        base = wid * TPW
        pltpu.sync_copy(g1_hbm.at[wid], i1_v)
        pltpu.sync_copy(g2_hbm.at[wid], i2_v)
        pltpu.sync_copy(w1_hbm.at[wid], w1_v)
        pltpu.sync_copy(w2_hbm.at[wid], w2_v)

        def start_gathers(ch):
            s = ch % 2
            sl = pl.ds(ch * CCH, CCH)
            return (pltpu.async_copy(y_hbm.at[i1_v.at[sl]], a_v[s], sa[s]),
                    pltpu.async_copy(y_hbm.at[i2_v.at[sl]], b_v[s], sb[s]))

        pending = {0: start_gathers(0)}
        stores = {}
        for ch in range(NCH):
            s = ch % 2
            if ch + 1 < NCH:
                pending[ch + 1] = start_gathers(ch + 1)
            ga, gb = pending.pop(ch)
            ga.wait()
            gb.wait()
            if ch >= 2:
                stores.pop(ch - 2).wait()

            def token_body(t, _, s=s, ch=ch):
                g1 = w1_v[ch * CCH + t]
                g2 = w2_v[ch * CCH + t]
                for j in range(HIDDEN // 16):
                    sl = pl.ds(j * 16, 16)
                    o_v[s][t, sl] = g1 * a_v[s][t, sl] + g2 * b_v[s][t, sl]
                return 0

            lax.fori_loop(0, CCH, token_body, 0)
            stores[ch] = pltpu.async_copy(
                o_v[s], out_hbm.at[pl.ds(base + ch * CCH, CCH)], so[s])
        for ch in sorted(stores):
            stores.pop(ch).wait()

    return _combine


def kernel(hidden_states, Wg, W1, W2):
    s1, s2, g1, g2, w1b, w2b, stats, laux = _gating(hidden_states, Wg)
    s1r = s1.reshape(NW, TPW)
    s2r = s2.reshape(NW, TPW)
    g1r = g1.reshape(NW, TPW)
    g2r = g2.reshape(NW, TPW)
    w1r = w1b.reshape(NW, TPW, 16)
    w2r = w2b.reshape(NW, TPW, 16)
    buf = _dispatch_kernel()(hidden_states, s1r, s2r)
    y = _ffn(stats, buf, W1, W2)
    out = _combine_kernel()(y, g1r, g2r, w1r, w2r)
    return (out, laux[0, 0], stats[1])
